# Initial kernel scaffold; baseline (speedup 1.0000x reference)
import jax
import jax.numpy as jnp
from jax.experimental import pallas as pl


def kernel(x, edge_index, W_in, b_in, gat_W, gat_att_src, gat_att_dst, gat_bias, lstm_Wih, lstm_Whh, W_out, b_out):
    raise NotImplementedError("write your pallas kernel here")


# speedup vs baseline: 45.6633x; 45.6633x over previous
import jax
import jax.numpy as jnp
from jax import lax
from jax.experimental import pallas as pl
from jax.experimental.pallas import tpu as pltpu
from jax.experimental.pallas import tpu_sc as plsc

N_NODES = 100000
D_FEAT = 128
HID = 32
OUT_DIM = 64
N_HOPS = 3

B_PRE = 1024
NTAB = 100352
PAD_DST = N_NODES

E2 = 1_600_000 + N_NODES
E2P = 1_703_936
KA = 1024
SLAB_A = E2P // 32
CHUNKS_A = SLAB_A // KA
KB = 512
SLAB_B = E2P // 16
CHUNKS_B = SLAB_B // KB
NHALF = 50000
NDUM = 1024
NLOC = 51200
TILES = 16
ROWS_T = NLOC // TILES
B_POST = 1000


def _pre_body(x_ref, w_ref, b_ref, uv_ref, xh_ref, asd_ref):
    xb = x_ref[...]
    xh = lax.dot_general(xb, w_ref[...], (((1,), (1,)), ((), ())),
                         preferred_element_type=jnp.float32) + b_ref[...]
    xh_ref[...] = xh
    asd_ref[...] = lax.dot_general(uv_ref[...], xh, (((1,), (1,)), ((), ())),
                                   preferred_element_type=jnp.float32)


def _sca_body(src_ref, dst_ref, asd_ref, e_ref,
              as_sp, ad_sp, srcb, dstb, asg, adg, ebuf):
    c = lax.axis_index("c")
    s = lax.axis_index("s")
    wid = c * TILES + s
    for h in range(N_HOPS):
        @pl.when(s == 0)
        def _():
            pltpu.sync_copy(asd_ref.at[h], as_sp)
            pltpu.sync_copy(asd_ref.at[N_HOPS + h], ad_sp)
        plsc.subcore_barrier()

        def chunk(q, carry):
            off = wid * SLAB_A + q * KA
            pltpu.sync_copy(src_ref.at[pl.ds(off, KA)], srcb)
            pltpu.sync_copy(dst_ref.at[pl.ds(off, KA)], dstb)
            pltpu.sync_copy(as_sp.at[srcb], asg)
            pltpu.sync_copy(ad_sp.at[dstb], adg)

            def ecomp(j, carry2):
                sl = pl.ds(j * 16, 16)
                av = asg[sl] + adg[sl]
                av = jnp.maximum(av, 0.2 * av)
                av = jnp.minimum(av, 60.0)
                ebuf[sl] = jnp.exp(av)
                return carry2
            lax.fori_loop(0, KA // 16, ecomp, 0)
            pltpu.sync_copy(ebuf, e_ref.at[h, pl.ds(off, KA)])
            return carry
        lax.fori_loop(0, CHUNKS_A, chunk, 0)
        plsc.subcore_barrier()


def _scb_body(src_ref, dst_ref, e_ref, xh_ref, out_ref,
              S_sp, Sd_sp, srcb, dstb, eb, rows, sem):
    c = lax.axis_index("c")
    s = lax.axis_index("s")
    base = c * NHALF
    zv = jnp.zeros((16,), jnp.float32)

    for h in range(N_HOPS):
        def zb_init(r, carry):
            rows[r, 0:16] = zv
            rows[r, 16:32] = zv
            return carry
        lax.fori_loop(0, 64, zb_init, 0)

        def zsd_init(r, carry):
            eb[pl.ds(r * 16, 16)] = zv
            return carry
        lax.fori_loop(0, 4, zsd_init, 0)

        def zero_body(r, carry):
            off = s * ROWS_T + r * 64
            pltpu.sync_copy(rows.at[pl.ds(0, 64)], S_sp.at[pl.ds(off, 64)])
            pltpu.sync_copy(eb.at[pl.ds(0, 64)], Sd_sp.at[pl.ds(off, 64)])
            return carry
        lax.fori_loop(0, ROWS_T // 64, zero_body, 0)

        plsc.subcore_barrier()

        def chunk(q, carry):
            off = s * SLAB_B + q * KB
            pltpu.sync_copy(src_ref.at[pl.ds(off, KB)], srcb)
            pltpu.sync_copy(dst_ref.at[pl.ds(off, KB)], dstb)
            cp = pltpu.async_copy(xh_ref.at[srcb], rows, sem)
            pltpu.sync_copy(e_ref.at[h, pl.ds(off, KB)], eb)

            def loccomp(j, carry2):
                sl = pl.ds(j * 16, 16)
                loc = dstb[sl] - base
                ok = (loc >= 0) & (loc < NHALF)
                dum = NHALF + (srcb[sl] & (NDUM - 1))
                dstb[sl] = jnp.where(ok, loc, dum)
                return carry2
            lax.fori_loop(0, KB // 16, loccomp, 0)

            cp.wait()

            def scale(j, carry2):
                ev_vec = eb[pl.ds(j * 16, 16)]
                for t in range(16):
                    k = j * 16 + t
                    ev = ev_vec[t]
                    rows[k, 0:16] = rows[k, 0:16] * ev
                    rows[k, 16:32] = rows[k, 16:32] * ev
                return carry2
            lax.fori_loop(0, KB // 16, scale, 0)

            pltpu.sync_copy(rows, S_sp.at[dstb], add=True)
            pltpu.sync_copy(eb, Sd_sp.at[dstb], add=True)
            return carry
        lax.fori_loop(0, CHUNKS_B, chunk, 0)

        plsc.subcore_barrier()

        for rc_off, rc_sz in ((0, KB), (KB, KB), (2 * KB, KB), (3 * KB, KB),
                              (4 * KB, KB), (5 * KB, KB),
                              (6 * KB, ROWS_T - 6 * KB)):
            roff = s * ROWS_T + rc_off
            pltpu.sync_copy(S_sp.at[pl.ds(roff, rc_sz)],
                            rows.at[pl.ds(0, rc_sz)])
            pltpu.sync_copy(Sd_sp.at[pl.ds(roff, rc_sz)],
                            eb.at[pl.ds(0, rc_sz)])

            def recip(j, carry):
                sl = pl.ds(j * 16, 16)
                eb[sl] = 1.0 / (eb[sl] + 1e-16)
                return carry
            lax.fori_loop(0, rc_sz // 16, recip, 0)

            def norm(j, carry):
                rv = eb[pl.ds(j * 16, 16)]
                for t in range(16):
                    r = j * 16 + t
                    rows[r, 0:16] = rows[r, 0:16] * rv[t]
                    rows[r, 16:32] = rows[r, 16:32] * rv[t]
                return carry
            lax.fori_loop(0, rc_sz // 16, norm, 0)

            pltpu.sync_copy(rows.at[pl.ds(0, rc_sz)],
                            out_ref.at[h, c, pl.ds(roff, rc_sz)])
        plsc.subcore_barrier()


def _post_body(sbar_ref, xh_ref, gw_ref, gb_ref, wih_ref, whh_ref, wo_ref,
               bo_ref, o_ref):
    xcur = xh_ref[...]
    h = jnp.zeros((B_POST, HID), jnp.float32)
    c = jnp.zeros((B_POST, HID), jnp.float32)
    for i in range(N_HOPS):
        sb = sbar_ref[i, 0]
        numer = lax.dot_general(sb, gw_ref[i], (((1,), (1,)), ((), ())),
                                preferred_element_type=jnp.float32)
        ht = jnp.tanh(numer + gb_ref[i])
        cat = jnp.concatenate([ht, xcur], axis=1)
        g = (lax.dot_general(cat, wih_ref[i], (((1,), (1,)), ((), ())),
                             preferred_element_type=jnp.float32)
             + lax.dot_general(h, whh_ref[i], (((1,), (1,)), ((), ())),
                               preferred_element_type=jnp.float32))
        gi = jax.nn.sigmoid(g[:, 0:HID])
        gf = jax.nn.sigmoid(g[:, HID:2 * HID])
        gg = jnp.tanh(g[:, 2 * HID:3 * HID])
        go = jax.nn.sigmoid(g[:, 3 * HID:4 * HID])
        c = gf * c + gi * gg
        h = go * jnp.tanh(c)
        xcur = h
    o_ref[...] = lax.dot_general(xcur, wo_ref[...], (((1,), (1,)), ((), ())),
                                 preferred_element_type=jnp.float32) + bo_ref[...]


def kernel(x, edge_index, W_in, b_in, gat_W, gat_att_src, gat_att_dst,
           gat_bias, lstm_Wih, lstm_Whh, W_out, b_out):
    loop = jnp.arange(N_NODES, dtype=jnp.int32)
    src = jnp.concatenate([edge_index[0].astype(jnp.int32), loop])
    dst = jnp.concatenate([edge_index[1].astype(jnp.int32), loop])
    src = jnp.pad(src, (0, E2P - E2), constant_values=0)
    dst = jnp.pad(dst, (0, E2P - E2), constant_values=PAD_DST)

    u = jnp.einsum("hij,hi->hj", gat_W, gat_att_src)
    v = jnp.einsum("hij,hi->hj", gat_W, gat_att_dst)
    uv = jnp.zeros((8, HID), jnp.float32).at[0:3].set(u).at[3:6].set(v)

    n_pre = NTAB // B_PRE
    xh, asd = pl.pallas_call(
        _pre_body,
        grid=(n_pre,),
        in_specs=[
            pl.BlockSpec((B_PRE, D_FEAT), lambda i: (i, 0)),
            pl.BlockSpec((HID, D_FEAT), lambda i: (0, 0)),
            pl.BlockSpec((1, HID), lambda i: (0, 0)),
            pl.BlockSpec((8, HID), lambda i: (0, 0)),
        ],
        out_specs=[
            pl.BlockSpec((B_PRE, HID), lambda i: (i, 0)),
            pl.BlockSpec((8, B_PRE), lambda i: (0, i)),
        ],
        out_shape=[
            jax.ShapeDtypeStruct((N_NODES, HID), jnp.float32),
            jax.ShapeDtypeStruct((8, NTAB), jnp.float32),
        ],
    )(x, W_in, b_in[None, :], uv)

    mesh = plsc.VectorSubcoreMesh(core_axis_name="c", subcore_axis_name="s")

    sca = pl.kernel(
        _sca_body,
        out_type=jax.ShapeDtypeStruct((N_HOPS, E2P), jnp.float32),
        mesh=mesh,
        compiler_params=pltpu.CompilerParams(use_tc_tiling_on_sc=False),
        scratch_types=[
            pltpu.VMEM_SHARED((NTAB,), jnp.float32),
            pltpu.VMEM_SHARED((NTAB,), jnp.float32),
            pltpu.VMEM((KA,), jnp.int32),
            pltpu.VMEM((KA,), jnp.int32),
            pltpu.VMEM((KA,), jnp.float32),
            pltpu.VMEM((KA,), jnp.float32),
            pltpu.VMEM((KA,), jnp.float32),
        ],
    )
    ew = sca(src, dst, asd)

    scb = pl.kernel(
        _scb_body,
        out_type=jax.ShapeDtypeStruct((N_HOPS, 2, NLOC, HID), jnp.float32),
        mesh=mesh,
        compiler_params=pltpu.CompilerParams(use_tc_tiling_on_sc=False),
        scratch_types=[
            pltpu.VMEM_SHARED((NLOC, HID), jnp.float32),
            pltpu.VMEM_SHARED((NLOC,), jnp.float32),
            pltpu.VMEM((KB,), jnp.int32),
            pltpu.VMEM((KB,), jnp.int32),
            pltpu.VMEM((KB,), jnp.float32),
            pltpu.VMEM((KB, HID), jnp.float32),
            pltpu.SemaphoreType.DMA,
        ],
    )
    sbar = scb(src, dst, ew, xh)

    out = pl.pallas_call(
        _post_body,
        grid=(N_NODES // B_POST,),
        in_specs=[
            pl.BlockSpec((N_HOPS, 1, B_POST, HID),
                         lambda i: (0, i // (NHALF // B_POST),
                                    i % (NHALF // B_POST), 0)),
            pl.BlockSpec((B_POST, HID), lambda i: (i, 0)),
            pl.BlockSpec((N_HOPS, HID, HID), lambda i: (0, 0, 0)),
            pl.BlockSpec((N_HOPS, 1, HID), lambda i: (0, 0, 0)),
            pl.BlockSpec((N_HOPS, 4 * HID, 2 * HID), lambda i: (0, 0, 0)),
            pl.BlockSpec((N_HOPS, 4 * HID, HID), lambda i: (0, 0, 0)),
            pl.BlockSpec((OUT_DIM, HID), lambda i: (0, 0)),
            pl.BlockSpec((1, OUT_DIM), lambda i: (0, 0)),
        ],
        out_specs=pl.BlockSpec((B_POST, OUT_DIM), lambda i: (i, 0)),
        out_shape=jax.ShapeDtypeStruct((N_NODES, OUT_DIM), jnp.float32),
    )(sbar, xh, gat_W, gat_bias[:, None, :], lstm_Wih, lstm_Whh, W_out,
      b_out[None, :])
    return out


# speedup vs baseline: 53.0172x; 1.1610x over previous
import jax
import jax.numpy as jnp
from jax import lax
from jax.experimental import pallas as pl
from jax.experimental.pallas import tpu as pltpu
from jax.experimental.pallas import tpu_sc as plsc

N_NODES = 100000
D_FEAT = 128
HID = 32
OUT_DIM = 64
N_HOPS = 3

B_PRE = 1024
NTAB = 100352
PAD_DST = N_NODES

E2 = 1_600_000 + N_NODES
E2P = 1_703_936
KA = 1024
SLAB_A = E2P // 32
CHUNKS_A = SLAB_A // KA
KB = 256
SLAB_B = E2P // 16
CHUNKS_B = SLAB_B // KB
NHALF = 50000
NDUM = 1024
NLOC = 51200
TILES = 16
ROWS_T = NLOC // TILES
B_POST = 1000


def _pre_body(x_ref, w_ref, b_ref, uv_ref, xh_ref, asd_ref):
    xb = x_ref[...]
    xh = lax.dot_general(xb, w_ref[...], (((1,), (1,)), ((), ())),
                         preferred_element_type=jnp.float32) + b_ref[...]
    xh_ref[...] = xh
    asd_ref[...] = lax.dot_general(uv_ref[...], xh, (((1,), (1,)), ((), ())),
                                   preferred_element_type=jnp.float32)


def _sca_body(src_ref, dst_ref, asd_ref, e_ref,
              as_sp, ad_sp, srcb, dstb, asg, adg, ebuf):
    c = lax.axis_index("c")
    s = lax.axis_index("s")
    wid = c * TILES + s
    for h in range(N_HOPS):
        @pl.when(s == 0)
        def _():
            pltpu.sync_copy(asd_ref.at[h], as_sp)
            pltpu.sync_copy(asd_ref.at[N_HOPS + h], ad_sp)
        plsc.subcore_barrier()

        def chunk(q, carry):
            off = wid * SLAB_A + q * KA
            pltpu.sync_copy(src_ref.at[pl.ds(off, KA)], srcb)
            pltpu.sync_copy(dst_ref.at[pl.ds(off, KA)], dstb)
            pltpu.sync_copy(as_sp.at[srcb], asg)
            pltpu.sync_copy(ad_sp.at[dstb], adg)

            def ecomp(j, carry2):
                sl = pl.ds(j * 16, 16)
                av = asg[sl] + adg[sl]
                av = jnp.maximum(av, 0.2 * av)
                av = jnp.minimum(av, 60.0)
                ebuf[sl] = jnp.exp(av)
                return carry2
            lax.fori_loop(0, KA // 16, ecomp, 0)
            pltpu.sync_copy(ebuf, e_ref.at[h, pl.ds(off, KA)])
            return carry
        lax.fori_loop(0, CHUNKS_A, chunk, 0)
        plsc.subcore_barrier()


def _scb_body(src_ref, dst_ref, e_ref, xh_ref, out_ref,
              S_sp, Sd_sp,
              srcb0, dstb0, eb0, rows0, srcb1, dstb1, eb1, rows1,
              isem0, isem1, gsem0, gsem1):
    c = lax.axis_index("c")
    s = lax.axis_index("s")
    base = c * NHALF
    zv = jnp.zeros((16,), jnp.float32)
    slots = ((srcb0, dstb0, eb0, rows0, isem0, gsem0),
             (srcb1, dstb1, eb1, rows1, isem1, gsem1))

    def issue_idx(b, off, h):
        srcb, dstb, eb, _, isem, _ = slots[b]
        pltpu.async_copy(src_ref.at[pl.ds(off, KB)], srcb, isem)
        pltpu.async_copy(dst_ref.at[pl.ds(off, KB)], dstb, isem)
        pltpu.async_copy(e_ref.at[h, pl.ds(off, KB)], eb, isem)

    def wait_idx(b):
        srcb, dstb, eb, _, isem, _ = slots[b]
        pltpu.make_async_copy(src_ref.at[pl.ds(0, KB)], srcb, isem).wait()
        pltpu.make_async_copy(dst_ref.at[pl.ds(0, KB)], dstb, isem).wait()
        pltpu.make_async_copy(e_ref.at[0, pl.ds(0, KB)], eb, isem).wait()

    def issue_gather(b):
        srcb, _, _, rows, _, gsem = slots[b]
        pltpu.async_copy(xh_ref.at[srcb], rows, gsem)

    def wait_gather(b):
        srcb, _, _, rows, _, gsem = slots[b]
        pltpu.make_async_copy(xh_ref.at[srcb], rows, gsem).wait()

    def compute_scatter(b):
        srcb, dstb, eb, rows, _, _ = slots[b]

        def loccomp(j, carry2):
            sl = pl.ds(j * 16, 16)
            loc = dstb[sl] - base
            ok = (loc >= 0) & (loc < NHALF)
            dum = NHALF + (srcb[sl] & (NDUM - 1))
            dstb[sl] = jnp.where(ok, loc, dum)
            return carry2
        lax.fori_loop(0, KB // 16, loccomp, 0)

        def scale(j, carry2):
            ev_vec = eb[pl.ds(j * 16, 16)]
            for t in range(16):
                k = j * 16 + t
                ev = ev_vec[t]
                rows[k, 0:16] = rows[k, 0:16] * ev
                rows[k, 16:32] = rows[k, 16:32] * ev
            return carry2
        lax.fori_loop(0, KB // 16, scale, 0)

        pltpu.sync_copy(rows, S_sp.at[dstb], add=True)
        pltpu.sync_copy(eb, Sd_sp.at[dstb], add=True)

    for h in range(N_HOPS):
        def zb_init(r, carry):
            rows0[r, 0:16] = zv
            rows0[r, 16:32] = zv
            return carry
        lax.fori_loop(0, 64, zb_init, 0)

        def zsd_init(r, carry):
            eb0[pl.ds(r * 16, 16)] = zv
            return carry
        lax.fori_loop(0, 4, zsd_init, 0)

        def zero_body(r, carry):
            off = s * ROWS_T + r * 64
            pltpu.sync_copy(rows0.at[pl.ds(0, 64)], S_sp.at[pl.ds(off, 64)])
            pltpu.sync_copy(eb0.at[pl.ds(0, 64)], Sd_sp.at[pl.ds(off, 64)])
            return carry
        lax.fori_loop(0, ROWS_T // 64, zero_body, 0)

        plsc.subcore_barrier()

        tile_off = s * SLAB_B

        issue_idx(0, tile_off, h)
        issue_idx(1, tile_off + KB, h)
        wait_idx(0)
        issue_gather(0)

        def pair(g, carry):
            q0 = 2 * g
            wait_idx(1)
            issue_gather(1)
            wait_gather(0)
            compute_scatter(0)
            issue_idx(0, tile_off + (q0 + 2) * KB, h)
            wait_idx(0)
            issue_gather(0)
            wait_gather(1)
            compute_scatter(1)
            issue_idx(1, tile_off + (q0 + 3) * KB, h)
            return carry
        lax.fori_loop(0, CHUNKS_B // 2 - 1, pair, 0)

        wait_idx(1)
        issue_gather(1)
        wait_gather(0)
        compute_scatter(0)
        wait_gather(1)
        compute_scatter(1)

        plsc.subcore_barrier()

        def norm_chunk(rc_off, rc_sz):
            roff = s * ROWS_T + rc_off
            pltpu.sync_copy(S_sp.at[pl.ds(roff, rc_sz)],
                            rows0.at[pl.ds(0, rc_sz)])
            pltpu.sync_copy(Sd_sp.at[pl.ds(roff, rc_sz)],
                            eb0.at[pl.ds(0, rc_sz)])

            def recip(j, carry):
                sl = pl.ds(j * 16, 16)
                eb0[sl] = 1.0 / (eb0[sl] + 1e-16)
                return carry
            lax.fori_loop(0, rc_sz // 16, recip, 0)

            def norm(j, carry):
                rv = eb0[pl.ds(j * 16, 16)]
                for t in range(16):
                    r = j * 16 + t
                    rows0[r, 0:16] = rows0[r, 0:16] * rv[t]
                    rows0[r, 16:32] = rows0[r, 16:32] * rv[t]
                return carry
            lax.fori_loop(0, rc_sz // 16, norm, 0)

            pltpu.sync_copy(rows0.at[pl.ds(0, rc_sz)],
                            out_ref.at[h, c, pl.ds(roff, rc_sz)])

        for r in range(ROWS_T // KB):
            norm_chunk(r * KB, KB)
        if ROWS_T % KB:
            norm_chunk((ROWS_T // KB) * KB, ROWS_T % KB)
        plsc.subcore_barrier()


def _post_body(sbar_ref, xh_ref, gw_ref, gb_ref, wih_ref, whh_ref, wo_ref,
               bo_ref, o_ref):
    xcur = xh_ref[...]
    h = jnp.zeros((B_POST, HID), jnp.float32)
    c = jnp.zeros((B_POST, HID), jnp.float32)
    for i in range(N_HOPS):
        sb = sbar_ref[i, 0]
        numer = lax.dot_general(sb, gw_ref[i], (((1,), (1,)), ((), ())),
                                preferred_element_type=jnp.float32)
        ht = jnp.tanh(numer + gb_ref[i])
        cat = jnp.concatenate([ht, xcur], axis=1)
        g = (lax.dot_general(cat, wih_ref[i], (((1,), (1,)), ((), ())),
                             preferred_element_type=jnp.float32)
             + lax.dot_general(h, whh_ref[i], (((1,), (1,)), ((), ())),
                               preferred_element_type=jnp.float32))
        gi = jax.nn.sigmoid(g[:, 0:HID])
        gf = jax.nn.sigmoid(g[:, HID:2 * HID])
        gg = jnp.tanh(g[:, 2 * HID:3 * HID])
        go = jax.nn.sigmoid(g[:, 3 * HID:4 * HID])
        c = gf * c + gi * gg
        h = go * jnp.tanh(c)
        xcur = h
    o_ref[...] = lax.dot_general(xcur, wo_ref[...], (((1,), (1,)), ((), ())),
                                 preferred_element_type=jnp.float32) + bo_ref[...]


def kernel(x, edge_index, W_in, b_in, gat_W, gat_att_src, gat_att_dst,
           gat_bias, lstm_Wih, lstm_Whh, W_out, b_out):
    loop = jnp.arange(N_NODES, dtype=jnp.int32)
    src = jnp.concatenate([edge_index[0].astype(jnp.int32), loop])
    dst = jnp.concatenate([edge_index[1].astype(jnp.int32), loop])
    src = jnp.pad(src, (0, E2P - E2), constant_values=0)
    dst = jnp.pad(dst, (0, E2P - E2), constant_values=PAD_DST)

    u = jnp.einsum("hij,hi->hj", gat_W, gat_att_src)
    v = jnp.einsum("hij,hi->hj", gat_W, gat_att_dst)
    uv = jnp.zeros((8, HID), jnp.float32).at[0:3].set(u).at[3:6].set(v)

    n_pre = NTAB // B_PRE
    xh, asd = pl.pallas_call(
        _pre_body,
        grid=(n_pre,),
        in_specs=[
            pl.BlockSpec((B_PRE, D_FEAT), lambda i: (i, 0)),
            pl.BlockSpec((HID, D_FEAT), lambda i: (0, 0)),
            pl.BlockSpec((1, HID), lambda i: (0, 0)),
            pl.BlockSpec((8, HID), lambda i: (0, 0)),
        ],
        out_specs=[
            pl.BlockSpec((B_PRE, HID), lambda i: (i, 0)),
            pl.BlockSpec((8, B_PRE), lambda i: (0, i)),
        ],
        out_shape=[
            jax.ShapeDtypeStruct((N_NODES, HID), jnp.float32),
            jax.ShapeDtypeStruct((8, NTAB), jnp.float32),
        ],
    )(x, W_in, b_in[None, :], uv)

    mesh = plsc.VectorSubcoreMesh(core_axis_name="c", subcore_axis_name="s")

    sca = pl.kernel(
        _sca_body,
        out_type=jax.ShapeDtypeStruct((N_HOPS, E2P), jnp.float32),
        mesh=mesh,
        compiler_params=pltpu.CompilerParams(use_tc_tiling_on_sc=False),
        scratch_types=[
            pltpu.VMEM_SHARED((NTAB,), jnp.float32),
            pltpu.VMEM_SHARED((NTAB,), jnp.float32),
            pltpu.VMEM((KA,), jnp.int32),
            pltpu.VMEM((KA,), jnp.int32),
            pltpu.VMEM((KA,), jnp.float32),
            pltpu.VMEM((KA,), jnp.float32),
            pltpu.VMEM((KA,), jnp.float32),
        ],
    )
    ew = sca(src, dst, asd)

    scb = pl.kernel(
        _scb_body,
        out_type=jax.ShapeDtypeStruct((N_HOPS, 2, NLOC, HID), jnp.float32),
        mesh=mesh,
        compiler_params=pltpu.CompilerParams(use_tc_tiling_on_sc=False),
        scratch_types=[
            pltpu.VMEM_SHARED((NLOC, HID), jnp.float32),
            pltpu.VMEM_SHARED((NLOC,), jnp.float32),
            pltpu.VMEM((KB,), jnp.int32),
            pltpu.VMEM((KB,), jnp.int32),
            pltpu.VMEM((KB,), jnp.float32),
            pltpu.VMEM((KB, HID), jnp.float32),
            pltpu.VMEM((KB,), jnp.int32),
            pltpu.VMEM((KB,), jnp.int32),
            pltpu.VMEM((KB,), jnp.float32),
            pltpu.VMEM((KB, HID), jnp.float32),
            pltpu.SemaphoreType.DMA,
            pltpu.SemaphoreType.DMA,
            pltpu.SemaphoreType.DMA,
            pltpu.SemaphoreType.DMA,
        ],
    )
    sbar = scb(src, dst, ew, xh)

    out = pl.pallas_call(
        _post_body,
        grid=(N_NODES // B_POST,),
        in_specs=[
            pl.BlockSpec((N_HOPS, 1, B_POST, HID),
                         lambda i: (0, i // (NHALF // B_POST),
                                    i % (NHALF // B_POST), 0)),
            pl.BlockSpec((B_POST, HID), lambda i: (i, 0)),
            pl.BlockSpec((N_HOPS, HID, HID), lambda i: (0, 0, 0)),
            pl.BlockSpec((N_HOPS, 1, HID), lambda i: (0, 0, 0)),
            pl.BlockSpec((N_HOPS, 4 * HID, 2 * HID), lambda i: (0, 0, 0)),
            pl.BlockSpec((N_HOPS, 4 * HID, HID), lambda i: (0, 0, 0)),
            pl.BlockSpec((OUT_DIM, HID), lambda i: (0, 0)),
            pl.BlockSpec((1, OUT_DIM), lambda i: (0, 0)),
        ],
        out_specs=pl.BlockSpec((B_POST, OUT_DIM), lambda i: (i, 0)),
        out_shape=jax.ShapeDtypeStruct((N_NODES, OUT_DIM), jnp.float32),
    )(sbar, xh, gat_W, gat_bias[:, None, :], lstm_Wih, lstm_Whh, W_out,
      b_out[None, :])
    return out


# speedup vs baseline: 53.0563x; 1.0007x over previous
import jax
import jax.numpy as jnp
from jax import lax
from jax.experimental import pallas as pl
from jax.experimental.pallas import tpu as pltpu
from jax.experimental.pallas import tpu_sc as plsc

N_NODES = 100000
D_FEAT = 128
HID = 32
OUT_DIM = 64
N_HOPS = 3

B_PRE = 1024
NTAB = 100352
PAD_DST = N_NODES

E2 = 1_600_000 + N_NODES
E2P = 1_703_936
KA = 1024
SLAB_A = E2P // 32
CHUNKS_A = SLAB_A // KA
KB = 256
SLAB_B = E2P // 16
CHUNKS_B = SLAB_B // KB
NHALF = 50000
NDUM = 1024
NLOC = 51200
TILES = 16
ROWS_T = NLOC // TILES
NRM = 64
B_POST = 1000


def _pre_body(x_ref, w_ref, b_ref, uv_ref, xh_ref, asd_ref):
    xb = x_ref[...]
    xh = lax.dot_general(xb, w_ref[...], (((1,), (1,)), ((), ())),
                         preferred_element_type=jnp.float32) + b_ref[...]
    xh_ref[...] = xh
    asd_ref[...] = lax.dot_general(uv_ref[...], xh, (((1,), (1,)), ((), ())),
                                   preferred_element_type=jnp.float32)


def _sca_body(src_ref, dst_ref, asd_ref, e_ref,
              as_sp, ad_sp, srcb, dstb, asg, adg, ebuf):
    c = lax.axis_index("c")
    s = lax.axis_index("s")
    wid = c * TILES + s
    for h in range(N_HOPS):
        @pl.when(s == 0)
        def _():
            pltpu.sync_copy(asd_ref.at[h], as_sp)
            pltpu.sync_copy(asd_ref.at[N_HOPS + h], ad_sp)
        plsc.subcore_barrier()

        def chunk(q, carry):
            off = wid * SLAB_A + q * KA
            pltpu.sync_copy(src_ref.at[pl.ds(off, KA)], srcb)
            pltpu.sync_copy(dst_ref.at[pl.ds(off, KA)], dstb)
            pltpu.sync_copy(as_sp.at[srcb], asg)
            pltpu.sync_copy(ad_sp.at[dstb], adg)

            def ecomp(j, carry2):
                sl = pl.ds(j * 16, 16)
                av = asg[sl] + adg[sl]
                av = jnp.maximum(av, 0.2 * av)
                av = jnp.minimum(av, 60.0)
                ebuf[sl] = jnp.exp(av)
                return carry2
            lax.fori_loop(0, KA // 16, ecomp, 0)
            pltpu.sync_copy(ebuf, e_ref.at[h, pl.ds(off, KA)])
            return carry
        lax.fori_loop(0, CHUNKS_A, chunk, 0)
        plsc.subcore_barrier()


def _scb_body(src_ref, dst_ref, e_ref, xh_ref, out_ref,
              S_sp, Sd_sp,
              srcb0, dstb0, eb0, rows0, srcb1, dstb1, eb1, rows1,
              isem0, isem1, gsem0, gsem1):
    c = lax.axis_index("c")
    s = lax.axis_index("s")
    base = c * NHALF
    zv = jnp.zeros((16,), jnp.float32)
    slots = ((srcb0, dstb0, eb0, rows0, isem0, gsem0),
             (srcb1, dstb1, eb1, rows1, isem1, gsem1))

    def issue_idx(b, off, h):
        srcb, dstb, eb, _, isem, _ = slots[b]
        pltpu.async_copy(src_ref.at[pl.ds(off, KB)], srcb, isem)
        pltpu.async_copy(dst_ref.at[pl.ds(off, KB)], dstb, isem)
        pltpu.async_copy(e_ref.at[h, pl.ds(off, KB)], eb, isem)

    def wait_idx(b):
        srcb, dstb, eb, _, isem, _ = slots[b]
        pltpu.make_async_copy(src_ref.at[pl.ds(0, KB)], srcb, isem).wait()
        pltpu.make_async_copy(dst_ref.at[pl.ds(0, KB)], dstb, isem).wait()
        pltpu.make_async_copy(e_ref.at[0, pl.ds(0, KB)], eb, isem).wait()

    def issue_gather(b):
        srcb, _, _, rows, _, gsem = slots[b]
        pltpu.async_copy(xh_ref.at[srcb], rows, gsem)

    def wait_gather(b):
        srcb, _, _, rows, _, gsem = slots[b]
        pltpu.make_async_copy(xh_ref.at[srcb], rows, gsem).wait()

    def compute_scatter(b):
        srcb, dstb, eb, rows, _, _ = slots[b]

        def loccomp(j, carry2):
            sl = pl.ds(j * 16, 16)
            loc = dstb[sl] - base
            ok = (loc >= 0) & (loc < NHALF)
            dum = NHALF + (srcb[sl] & (NDUM - 1))
            dstb[sl] = jnp.where(ok, loc, dum)
            return carry2
        lax.fori_loop(0, KB // 16, loccomp, 0)

        def scale(j, carry2):
            ev_vec = eb[pl.ds(j * 16, 16)]
            for t in range(16):
                k = j * 16 + t
                ev = ev_vec[t]
                rows[k, 0:16] = rows[k, 0:16] * ev
                rows[k, 16:32] = rows[k, 16:32] * ev
            return carry2
        lax.fori_loop(0, KB // 16, scale, 0)

        pltpu.sync_copy(rows, S_sp.at[dstb], add=True)
        pltpu.sync_copy(eb, Sd_sp.at[dstb], add=True)

    for h in range(N_HOPS):
        def zb_init(r, carry):
            rows0[r, 0:16] = zv
            rows0[r, 16:32] = zv
            return carry
        lax.fori_loop(0, NRM, zb_init, 0)

        def zsd_init(r, carry):
            eb0[pl.ds(r * 16, 16)] = zv
            return carry
        lax.fori_loop(0, NRM // 16, zsd_init, 0)

        def zero_body(r, carry):
            off = s * ROWS_T + r * NRM
            pltpu.sync_copy(rows0.at[pl.ds(0, NRM)], S_sp.at[pl.ds(off, NRM)])
            pltpu.sync_copy(eb0.at[pl.ds(0, NRM)], Sd_sp.at[pl.ds(off, NRM)])
            return carry
        lax.fori_loop(0, ROWS_T // NRM, zero_body, 0)

        plsc.subcore_barrier()

        tile_off = s * SLAB_B

        issue_idx(0, tile_off, h)
        issue_idx(1, tile_off + KB, h)
        wait_idx(0)
        issue_gather(0)

        def pair(g, carry):
            q0 = 2 * g
            wait_idx(1)
            issue_gather(1)
            wait_gather(0)
            compute_scatter(0)
            issue_idx(0, tile_off + (q0 + 2) * KB, h)
            wait_idx(0)
            issue_gather(0)
            wait_gather(1)
            compute_scatter(1)
            issue_idx(1, tile_off + (q0 + 3) * KB, h)
            return carry
        lax.fori_loop(0, CHUNKS_B // 2 - 1, pair, 0)

        wait_idx(1)
        issue_gather(1)
        wait_gather(0)
        compute_scatter(0)
        wait_gather(1)
        compute_scatter(1)

        plsc.subcore_barrier()

        def norm_chunk(rc_off, sz):
            roff = s * ROWS_T + rc_off
            pltpu.sync_copy(S_sp.at[pl.ds(roff, sz)], rows0.at[pl.ds(0, sz)])
            pltpu.sync_copy(Sd_sp.at[pl.ds(roff, sz)], eb0.at[pl.ds(0, sz)])

            def recip(j, carry):
                sl = pl.ds(j * 16, 16)
                eb0[sl] = 1.0 / (eb0[sl] + 1e-16)
                return carry
            lax.fori_loop(0, sz // 16, recip, 0)

            def norm(j, carry):
                rv = eb0[pl.ds(j * 16, 16)]
                for t in range(16):
                    r = j * 16 + t
                    rows0[r, 0:16] = rows0[r, 0:16] * rv[t]
                    rows0[r, 16:32] = rows0[r, 16:32] * rv[t]
                return carry
            lax.fori_loop(0, sz // 16, norm, 0)

            pltpu.sync_copy(rows0.at[pl.ds(0, sz)],
                            out_ref.at[h, c, pl.ds(roff, sz)])

        for rc in range(ROWS_T // KB):
            norm_chunk(rc * KB, KB)
        if ROWS_T % KB:
            norm_chunk((ROWS_T // KB) * KB, ROWS_T % KB)
        plsc.subcore_barrier()


def _post_body(sbar_ref, xh_ref, gw_ref, gb_ref, wih_ref, whh_ref, wo_ref,
               bo_ref, o_ref):
    xcur = xh_ref[...]
    h = jnp.zeros((B_POST, HID), jnp.float32)
    c = jnp.zeros((B_POST, HID), jnp.float32)
    for i in range(N_HOPS):
        sb = sbar_ref[i, 0]
        numer = lax.dot_general(sb, gw_ref[i], (((1,), (1,)), ((), ())),
                                preferred_element_type=jnp.float32)
        ht = jnp.tanh(numer + gb_ref[i])
        cat = jnp.concatenate([ht, xcur], axis=1)
        g = (lax.dot_general(cat, wih_ref[i], (((1,), (1,)), ((), ())),
                             preferred_element_type=jnp.float32)
             + lax.dot_general(h, whh_ref[i], (((1,), (1,)), ((), ())),
                               preferred_element_type=jnp.float32))
        gi = jax.nn.sigmoid(g[:, 0:HID])
        gf = jax.nn.sigmoid(g[:, HID:2 * HID])
        gg = jnp.tanh(g[:, 2 * HID:3 * HID])
        go = jax.nn.sigmoid(g[:, 3 * HID:4 * HID])
        c = gf * c + gi * gg
        h = go * jnp.tanh(c)
        xcur = h
    o_ref[...] = lax.dot_general(xcur, wo_ref[...], (((1,), (1,)), ((), ())),
                                 preferred_element_type=jnp.float32) + bo_ref[...]


def kernel(x, edge_index, W_in, b_in, gat_W, gat_att_src, gat_att_dst,
           gat_bias, lstm_Wih, lstm_Whh, W_out, b_out):
    loop = jnp.arange(N_NODES, dtype=jnp.int32)
    src = jnp.concatenate([edge_index[0].astype(jnp.int32), loop])
    dst = jnp.concatenate([edge_index[1].astype(jnp.int32), loop])
    src = jnp.pad(src, (0, E2P - E2), constant_values=0)
    dst = jnp.pad(dst, (0, E2P - E2), constant_values=PAD_DST)

    u = jnp.einsum("hij,hi->hj", gat_W, gat_att_src)
    v = jnp.einsum("hij,hi->hj", gat_W, gat_att_dst)
    uv = jnp.zeros((8, HID), jnp.float32).at[0:3].set(u).at[3:6].set(v)

    n_pre = NTAB // B_PRE
    xh, asd = pl.pallas_call(
        _pre_body,
        grid=(n_pre,),
        in_specs=[
            pl.BlockSpec((B_PRE, D_FEAT), lambda i: (i, 0)),
            pl.BlockSpec((HID, D_FEAT), lambda i: (0, 0)),
            pl.BlockSpec((1, HID), lambda i: (0, 0)),
            pl.BlockSpec((8, HID), lambda i: (0, 0)),
        ],
        out_specs=[
            pl.BlockSpec((B_PRE, HID), lambda i: (i, 0)),
            pl.BlockSpec((8, B_PRE), lambda i: (0, i)),
        ],
        out_shape=[
            jax.ShapeDtypeStruct((N_NODES, HID), jnp.float32),
            jax.ShapeDtypeStruct((8, NTAB), jnp.float32),
        ],
    )(x, W_in, b_in[None, :], uv)

    mesh = plsc.VectorSubcoreMesh(core_axis_name="c", subcore_axis_name="s")

    sca = pl.kernel(
        _sca_body,
        out_type=jax.ShapeDtypeStruct((N_HOPS, E2P), jnp.float32),
        mesh=mesh,
        compiler_params=pltpu.CompilerParams(use_tc_tiling_on_sc=False),
        scratch_types=[
            pltpu.VMEM_SHARED((NTAB,), jnp.float32),
            pltpu.VMEM_SHARED((NTAB,), jnp.float32),
            pltpu.VMEM((KA,), jnp.int32),
            pltpu.VMEM((KA,), jnp.int32),
            pltpu.VMEM((KA,), jnp.float32),
            pltpu.VMEM((KA,), jnp.float32),
            pltpu.VMEM((KA,), jnp.float32),
        ],
    )
    ew = sca(src, dst, asd)

    scb = pl.kernel(
        _scb_body,
        out_type=jax.ShapeDtypeStruct((N_HOPS, 2, NLOC, HID), jnp.float32),
        mesh=mesh,
        compiler_params=pltpu.CompilerParams(use_tc_tiling_on_sc=False),
        scratch_types=[
            pltpu.VMEM_SHARED((NLOC, HID), jnp.float32),
            pltpu.VMEM_SHARED((NLOC,), jnp.float32),
            pltpu.VMEM((KB,), jnp.int32),
            pltpu.VMEM((KB,), jnp.int32),
            pltpu.VMEM((KB,), jnp.float32),
            pltpu.VMEM((KB, HID), jnp.float32),
            pltpu.VMEM((KB,), jnp.int32),
            pltpu.VMEM((KB,), jnp.int32),
            pltpu.VMEM((KB,), jnp.float32),
            pltpu.VMEM((KB, HID), jnp.float32),
            pltpu.SemaphoreType.DMA,
            pltpu.SemaphoreType.DMA,
            pltpu.SemaphoreType.DMA,
            pltpu.SemaphoreType.DMA,
        ],
    )
    sbar = scb(src, dst, ew, xh)

    out = pl.pallas_call(
        _post_body,
        grid=(N_NODES // B_POST,),
        in_specs=[
            pl.BlockSpec((N_HOPS, 1, B_POST, HID),
                         lambda i: (0, i // (NHALF // B_POST),
                                    i % (NHALF // B_POST), 0)),
            pl.BlockSpec((B_POST, HID), lambda i: (i, 0)),
            pl.BlockSpec((N_HOPS, HID, HID), lambda i: (0, 0, 0)),
            pl.BlockSpec((N_HOPS, 1, HID), lambda i: (0, 0, 0)),
            pl.BlockSpec((N_HOPS, 4 * HID, 2 * HID), lambda i: (0, 0, 0)),
            pl.BlockSpec((N_HOPS, 4 * HID, HID), lambda i: (0, 0, 0)),
            pl.BlockSpec((OUT_DIM, HID), lambda i: (0, 0)),
            pl.BlockSpec((1, OUT_DIM), lambda i: (0, 0)),
        ],
        out_specs=pl.BlockSpec((B_POST, OUT_DIM), lambda i: (i, 0)),
        out_shape=jax.ShapeDtypeStruct((N_NODES, OUT_DIM), jnp.float32),
    )(sbar, xh, gat_W, gat_bias[:, None, :], lstm_Wih, lstm_Whh, W_out,
      b_out[None, :])
    return out
